# baseline (speedup 1.0000x reference)
import functools

import jax
import jax.numpy as jnp
from jax import lax
from jax.experimental import pallas as pl
from jax.experimental.pallas import tpu as pltpu
from jax.experimental.pallas import tpu_sc as plsc

VOCAB = 30522
HIDDEN = 768
MAX_POS = 512
TYPE_VOCAB = 2
BATCH = 64
SEQ = 512
EPS = 1e-12

NC, NS, L = 2, 16, 16
NW = NC * NS
SEQ_PER_W = BATCH // NW
CHUNK = 32
NCHUNK = SEQ // CHUNK
NSLICE = HIDDEN // L
UNROLL = 8

_mesh = plsc.VectorSubcoreMesh(core_axis_name="c", subcore_axis_name="s")


@functools.partial(
    pl.kernel,
    out_type=jax.ShapeDtypeStruct((BATCH * SEQ, HIDDEN), jnp.float32),
    mesh=_mesh,
    compiler_params=pltpu.CompilerParams(needs_layout_passes=False),
    scratch_types=[
        pltpu.VMEM((CHUNK,), jnp.int32),
        pltpu.VMEM((CHUNK,), jnp.int32),
        pltpu.VMEM((CHUNK, HIDDEN), jnp.float32),
        pltpu.VMEM((CHUNK, HIDDEN), jnp.float32),
        pltpu.VMEM((CHUNK, HIDDEN), jnp.float32),
        pltpu.VMEM((HIDDEN,), jnp.float32),
        pltpu.VMEM((HIDDEN,), jnp.float32),
        pltpu.SemaphoreType.DMA,
    ],
)
def _bert_emb_sc(ids_hbm, tts_hbm, word_hbm, pos_hbm, type_hbm, gamma_hbm,
                 beta_hbm, out_hbm, idx_v, tt_v, pos_v, rows_v, trows_v,
                 gamma_v, beta_v, sem):
    wid = lax.axis_index("s") * NC + lax.axis_index("c")

    pltpu.sync_copy(gamma_hbm, gamma_v)
    pltpu.sync_copy(beta_hbm, beta_v)

    inv_h = jnp.float32(1.0 / HIDDEN)

    def token_body(i, _):
        def acc_body(jo, carry):
            s, ss = carry
            for u in range(UNROLL):
                col = (jo * UNROLL + u) * L
                e = (rows_v[i, pl.ds(col, L)] + pos_v[i, pl.ds(col, L)]
                     + trows_v[i, pl.ds(col, L)])
                rows_v[i, pl.ds(col, L)] = e
                s = s + e
                ss = ss + e * e
            return s, ss

        z = jnp.zeros((L,), jnp.float32)
        s, ss = lax.fori_loop(0, NSLICE // UNROLL, acc_body, (z, z))
        mean = jnp.sum(s) * inv_h
        var = jnp.sum(ss) * inv_h - mean * mean
        x = jnp.full((L,), var + EPS, jnp.float32)
        xi = plsc.bitcast(x, jnp.int32)
        y = plsc.bitcast(jnp.int32(0x5F3759DF) - (xi >> 1), jnp.float32)
        half_x = x * 0.5
        for _ in range(3):
            y = y * (1.5 - half_x * y * y)
        mean_v = jnp.full((L,), mean, jnp.float32)

        def norm_body(jo, c):
            for u in range(UNROLL):
                col = (jo * UNROLL + u) * L
                e = rows_v[i, pl.ds(col, L)]
                g = gamma_v[pl.ds(col, L)]
                b = beta_v[pl.ds(col, L)]
                rows_v[i, pl.ds(col, L)] = (e - mean_v) * y * g + b
            return c

        lax.fori_loop(0, NSLICE // UNROLL, norm_body, 0)
        return 0

    def chunk_body(r, _):
        pltpu.sync_copy(pos_hbm.at[pl.ds(r * CHUNK, CHUNK)], pos_v)
        for q in range(SEQ_PER_W):
            base = (wid * SEQ_PER_W + q) * SEQ + r * CHUNK
            pltpu.sync_copy(ids_hbm.at[pl.ds(base, CHUNK)], idx_v)
            pltpu.sync_copy(tts_hbm.at[pl.ds(base, CHUNK)], tt_v)
            cw = pltpu.async_copy(word_hbm.at[idx_v], rows_v, sem)
            ct = pltpu.async_copy(type_hbm.at[tt_v], trows_v, sem)
            cw.wait()
            ct.wait()
            lax.fori_loop(0, CHUNK, token_body, 0)
            pltpu.sync_copy(rows_v, out_hbm.at[pl.ds(base, CHUNK)])
        return 0

    lax.fori_loop(0, NCHUNK, chunk_body, 0)


def kernel(input_ids, token_type_ids, word_embeddings, position_embeddings,
           token_type_embeddings, ln_gamma, ln_beta):
    ids = input_ids.reshape(-1).astype(jnp.int32)
    tts = token_type_ids.reshape(-1).astype(jnp.int32)
    out = _bert_emb_sc(ids, tts, word_embeddings, position_embeddings,
                       token_type_embeddings, ln_gamma, ln_beta)
    return out.reshape(BATCH, SEQ, HIDDEN)
